# Initial kernel scaffold; baseline (speedup 1.0000x reference)
import jax
import jax.numpy as jnp
from jax.experimental import pallas as pl


def kernel(x, edge_index, W1, a1s, a1d, b1, W2, a2s, a2d, b2, W3, a3s, a3d, b3, Wl1, bl1, Wl2, bl2, Wl3, bl3, Wl4, bl4):
    raise NotImplementedError("write your pallas kernel here")


# speedup vs baseline: 47.1147x; 47.1147x over previous
import functools

import jax
import jax.numpy as jnp
from jax import lax
from jax.experimental import pallas as pl
from jax.experimental.pallas import tpu as pltpu
from jax.experimental.pallas import tpu_sc as plsc

N = 10000
NP = 10240
E = 320000
ESL = E + N
NC, NS, L = 2, 16, 16
NW = NC * NS
CB = 128
EPAD = 331776
CE = EPAD // NW
CHUNKS = CE // CB
ROWS_PER = NP // NS
F = 16


def _proj_tail(h, as_w, ad_w, h_ref, asv_ref, adv_ref, cvec_ref):
    h_ref[...] = h
    asv = jnp.dot(h, as_w, preferred_element_type=jnp.float32)
    adv = jnp.dot(h, ad_w, preferred_element_type=jnp.float32)
    asv_ref[...] = asv
    adv_ref[...] = adv
    s = jnp.max(asv) + jnp.max(adv)
    c = jnp.maximum(s, 0.2 * s)
    cvec_ref[...] = jnp.full((L,), c, jnp.float32)


def _tc_front_body(x_ref, w_ref, as_ref, ad_ref,
                   h_ref, asv_ref, adv_ref, cvec_ref):
    h = jnp.dot(x_ref[...], w_ref[...], preferred_element_type=jnp.float32)
    _proj_tail(h, as_ref[...], ad_ref[...], h_ref, asv_ref, adv_ref, cvec_ref)


def _merge(num_ref, den_ref, b_ref):
    num = num_ref[0] + num_ref[1]
    den = den_ref[0] + den_ref[1]
    agg = num / (den[:, None] + 1e-16)
    g = agg + b_ref[...][None, :]
    return jnp.maximum(g, 0.2 * g)


def _tc_mid_body(num_ref, den_ref, b_ref, w_ref, as_ref, ad_ref,
                 h_ref, asv_ref, adv_ref, cvec_ref):
    g = _merge(num_ref, den_ref, b_ref)
    h = jnp.dot(g, w_ref[...], preferred_element_type=jnp.float32)
    _proj_tail(h, as_ref[...], ad_ref[...], h_ref, asv_ref, adv_ref, cvec_ref)


def _tc_final_body(num_ref, den_ref, b_ref,
                   w1_ref, b1_ref, w2_ref, b2_ref, w3_ref, b3_ref,
                   w4_ref, b4_ref, out_ref):
    g = _merge(num_ref, den_ref, b_ref)
    t = jnp.dot(g, w1_ref[...], preferred_element_type=jnp.float32) + b1_ref[...][None, :]
    t = jnp.maximum(t, 0.0)
    t = jnp.dot(t, w2_ref[...], preferred_element_type=jnp.float32) + b2_ref[...][None, :]
    t = jnp.maximum(t, 0.0)
    t = jnp.dot(t, w3_ref[...], preferred_element_type=jnp.float32) + b3_ref[...][None, :]
    t = jnp.maximum(t, 0.0)
    out_ref[...] = jnp.dot(t, w4_ref[...], preferred_element_type=jnp.float32) + b4_ref[...][None, :]


def _tc_front(x_pad, W, a_s, a_d):
    return pl.pallas_call(
        _tc_front_body,
        out_shape=(
            jax.ShapeDtypeStruct((NP, F), jnp.float32),
            jax.ShapeDtypeStruct((NP,), jnp.float32),
            jax.ShapeDtypeStruct((NP,), jnp.float32),
            jax.ShapeDtypeStruct((L,), jnp.float32),
        ),
    )(x_pad, W, a_s, a_d)


def _tc_mid(numP, denP, b, W, a_s, a_d):
    return pl.pallas_call(
        _tc_mid_body,
        out_shape=(
            jax.ShapeDtypeStruct((NP, F), jnp.float32),
            jax.ShapeDtypeStruct((NP,), jnp.float32),
            jax.ShapeDtypeStruct((NP,), jnp.float32),
            jax.ShapeDtypeStruct((L,), jnp.float32),
        ),
    )(numP, denP, b, W, a_s, a_d)


def _tc_final(numP, denP, b, Wl1, bl1, Wl2, bl2, Wl3, bl3, Wl4, bl4):
    return pl.pallas_call(
        _tc_final_body,
        out_shape=jax.ShapeDtypeStruct((NP, 128), jnp.float32),
    )(numP, denP, b, Wl1, bl1, Wl2, bl2, Wl3, bl3, Wl4, bl4)


def _sc_edges_body(src_hbm, dst_hbm, h_hbm, as_hbm, ad_hbm, cvec_hbm,
                   num_out, den_out,
                   as_v, ad_v, c_v, si_v, di_v, hr_v, wr_v, p_v,
                   num_sh, den_sh, sem):
    cid = lax.axis_index("c")
    sid = lax.axis_index("s")
    wid = cid * NS + sid

    pltpu.sync_copy(as_hbm, as_v)
    pltpu.sync_copy(ad_hbm, ad_v)
    pltpu.sync_copy(cvec_hbm, c_v)
    cval = c_v[...]

    zero16 = jnp.zeros((L,), jnp.float32)

    def _zrow(j, carry):
        wr_v[j, :] = zero16
        p_v[pl.ds((j % 8) * L, L)] = zero16
        return carry

    lax.fori_loop(0, CB, _zrow, 0)
    row0 = sid * ROWS_PER
    for k in range(ROWS_PER // CB):
        num_sh_slc = num_sh.at[pl.ds(row0 + k * CB, CB)]
        pltpu.sync_copy(wr_v, num_sh_slc)
        pltpu.sync_copy(p_v, den_sh.at[pl.ds(row0 + k * CB, CB)])
    plsc.subcore_barrier()

    base0 = wid * CE

    def _chunk(c, carry):
        base = base0 + c * CB
        pltpu.sync_copy(src_hbm.at[pl.ds(base, CB)], si_v)
        pltpu.sync_copy(dst_hbm.at[pl.ds(base, CB)], di_v)
        pltpu.async_copy(h_hbm.at[si_v], hr_v, sem).wait()
        for g in range(CB // L):
            siv = si_v[pl.ds(g * L, L)]
            div = di_v[pl.ds(g * L, L)]
            av = plsc.load_gather(as_v, [siv])
            dv = plsc.load_gather(ad_v, [div])
            s = av + dv
            e = jnp.maximum(s, 0.2 * s)
            p = jnp.exp(e - cval)
            p_v[pl.ds(g * L, L)] = p
            for j in range(L):
                pi = p[j]
                wr_v[g * L + j, :] = hr_v[g * L + j, :] * pi
        pltpu.sync_copy(wr_v, num_sh.at[di_v], add=True)
        pltpu.sync_copy(p_v, den_sh.at[di_v], add=True)
        return carry

    lax.fori_loop(0, CHUNKS, _chunk, 0)
    plsc.subcore_barrier()

    for k in range(ROWS_PER // CB):
        r = row0 + k * CB
        pltpu.sync_copy(num_sh.at[pl.ds(r, CB)], wr_v)
        pltpu.sync_copy(wr_v, num_out.at[cid].at[pl.ds(r, CB)])
        pltpu.sync_copy(den_sh.at[pl.ds(r, CB)], p_v)
        pltpu.sync_copy(p_v, den_out.at[cid].at[pl.ds(r, CB)])


_sc_edges = pl.kernel(
    _sc_edges_body,
    out_type=(
        jax.ShapeDtypeStruct((NC, NP, F), jnp.float32),
        jax.ShapeDtypeStruct((NC, NP), jnp.float32),
    ),
    mesh=plsc.VectorSubcoreMesh(core_axis_name="c", subcore_axis_name="s",
                                num_cores=NC, num_subcores=NS),
    scratch_types=[
        pltpu.VMEM((NP,), jnp.float32),
        pltpu.VMEM((NP,), jnp.float32),
        pltpu.VMEM((L,), jnp.float32),
        pltpu.VMEM((CB,), jnp.int32),
        pltpu.VMEM((CB,), jnp.int32),
        pltpu.VMEM((CB, F), jnp.float32),
        pltpu.VMEM((CB, F), jnp.float32),
        pltpu.VMEM((CB,), jnp.float32),
        pltpu.VMEM_SHARED((NP, F), jnp.float32),
        pltpu.VMEM_SHARED((NP,), jnp.float32),
        pltpu.SemaphoreType.DMA,
    ],
    compiler_params=pltpu.CompilerParams(needs_layout_passes=False,
                                         use_tc_tiling_on_sc=False),
)


def kernel(x, edge_index, W1, a1s, a1d, b1, W2, a2s, a2d, b2,
           W3, a3s, a3d, b3, Wl1, bl1, Wl2, bl2, Wl3, bl3, Wl4, bl4):
    loop = jnp.arange(N, dtype=jnp.int32)
    padi = jnp.full((EPAD - ESL,), N, dtype=jnp.int32)
    srcp = jnp.concatenate([edge_index[0], loop, padi])
    dstp = jnp.concatenate([edge_index[1], loop, padi])
    x_pad = jnp.pad(x, ((0, NP - N), (0, 0)))

    h, asv, adv, cvec = _tc_front(x_pad, W1, a1s, a1d)
    numP, denP = _sc_edges(srcp, dstp, h, asv, adv, cvec)
    h, asv, adv, cvec = _tc_mid(numP, denP, b1, W2, a2s, a2d)
    numP, denP = _sc_edges(srcp, dstp, h, asv, adv, cvec)
    h, asv, adv, cvec = _tc_mid(numP, denP, b2, W3, a3s, a3d)
    numP, denP = _sc_edges(srcp, dstp, h, asv, adv, cvec)
    out = _tc_final(numP, denP, b3, Wl1, bl1, Wl2, bl2, Wl3, bl3, Wl4, bl4)
    return out[:N]


# speedup vs baseline: 89.6975x; 1.9038x over previous
import functools

import jax
import jax.numpy as jnp
from jax import lax
from jax.experimental import pallas as pl
from jax.experimental.pallas import tpu as pltpu
from jax.experimental.pallas import tpu_sc as plsc

N = 10000
NP = 10240
E = 320000
ESL = E + N
NC, NS, L = 2, 16, 16
NW = NC * NS
CB = 128
CHUNKS = 82
CE = CHUNKS * CB
EPAD = NW * CE
ROWS_PER = NP // NS
F = 16


def _proj_tail(h, as_w, ad_w, h_ref, asv_ref, adv_ref, cvec_ref):
    h_ref[...] = h
    asv = jnp.dot(h, as_w, preferred_element_type=jnp.float32)
    adv = jnp.dot(h, ad_w, preferred_element_type=jnp.float32)
    asv_ref[...] = asv
    adv_ref[...] = adv
    s = jnp.max(asv) + jnp.max(adv)
    c = jnp.maximum(s, 0.2 * s)
    cvec_ref[...] = jnp.full((L,), c, jnp.float32)


def _tc_front_body(x_ref, w_ref, as_ref, ad_ref,
                   h_ref, asv_ref, adv_ref, cvec_ref):
    h = jnp.dot(x_ref[...], w_ref[...], preferred_element_type=jnp.float32)
    _proj_tail(h, as_ref[...], ad_ref[...], h_ref, asv_ref, adv_ref, cvec_ref)


def _merge(num_ref, den_ref, b_ref):
    num = num_ref[0] + num_ref[1]
    den = den_ref[0] + den_ref[1]
    agg = num / (den[:, None] + 1e-16)
    g = agg + b_ref[...][None, :]
    return jnp.maximum(g, 0.2 * g)


def _tc_mid_body(num_ref, den_ref, b_ref, w_ref, as_ref, ad_ref,
                 h_ref, asv_ref, adv_ref, cvec_ref):
    g = _merge(num_ref, den_ref, b_ref)
    h = jnp.dot(g, w_ref[...], preferred_element_type=jnp.float32)
    _proj_tail(h, as_ref[...], ad_ref[...], h_ref, asv_ref, adv_ref, cvec_ref)


def _tc_final_body(num_ref, den_ref, b_ref,
                   w1_ref, b1_ref, w2_ref, b2_ref, w3_ref, b3_ref,
                   w4_ref, b4_ref, out_ref):
    g = _merge(num_ref, den_ref, b_ref)
    t = jnp.dot(g, w1_ref[...], preferred_element_type=jnp.float32) + b1_ref[...][None, :]
    t = jnp.maximum(t, 0.0)
    t = jnp.dot(t, w2_ref[...], preferred_element_type=jnp.float32) + b2_ref[...][None, :]
    t = jnp.maximum(t, 0.0)
    t = jnp.dot(t, w3_ref[...], preferred_element_type=jnp.float32) + b3_ref[...][None, :]
    t = jnp.maximum(t, 0.0)
    out_ref[...] = jnp.dot(t, w4_ref[...], preferred_element_type=jnp.float32) + b4_ref[...][None, :]


def _tc_front(x_pad, W, a_s, a_d):
    return pl.pallas_call(
        _tc_front_body,
        out_shape=(
            jax.ShapeDtypeStruct((NP, F), jnp.float32),
            jax.ShapeDtypeStruct((NP,), jnp.float32),
            jax.ShapeDtypeStruct((NP,), jnp.float32),
            jax.ShapeDtypeStruct((L,), jnp.float32),
        ),
    )(x_pad, W, a_s, a_d)


def _tc_mid(numP, denP, b, W, a_s, a_d):
    return pl.pallas_call(
        _tc_mid_body,
        out_shape=(
            jax.ShapeDtypeStruct((NP, F), jnp.float32),
            jax.ShapeDtypeStruct((NP,), jnp.float32),
            jax.ShapeDtypeStruct((NP,), jnp.float32),
            jax.ShapeDtypeStruct((L,), jnp.float32),
        ),
    )(numP, denP, b, W, a_s, a_d)


def _tc_final(numP, denP, b, Wl1, bl1, Wl2, bl2, Wl3, bl3, Wl4, bl4):
    return pl.pallas_call(
        _tc_final_body,
        out_shape=jax.ShapeDtypeStruct((NP, 128), jnp.float32),
    )(numP, denP, b, Wl1, bl1, Wl2, bl2, Wl3, bl3, Wl4, bl4)


def _sc_edges_body(idx_hbm, h_hbm, as_hbm, ad_hbm, cvec_hbm,
                   num_out, den_out,
                   as_v, ad_v, c_v, idx_v, hr0, hr1, wr0, wr1, pb0, pb1,
                   num_sh, den_sh,
                   gsem0, gsem1, nsem0, nsem1, dsem0, dsem1):
    cid = lax.axis_index("c")
    sid = lax.axis_index("s")
    wid = cid * NS + sid

    pltpu.sync_copy(as_hbm, as_v)
    pltpu.sync_copy(ad_hbm, ad_v)
    pltpu.sync_copy(cvec_hbm, c_v)
    pltpu.sync_copy(idx_hbm.at[wid], idx_v)
    cval = c_v[...]

    zero16 = jnp.zeros((L,), jnp.float32)

    def _zrow(j, carry):
        wr0[j, :] = zero16
        pb0[pl.ds((j % 8) * L, L)] = zero16
        return carry

    lax.fori_loop(0, CB, _zrow, 0)
    row0 = sid * ROWS_PER
    for k in range(ROWS_PER // CB):
        pltpu.sync_copy(wr0, num_sh.at[pl.ds(row0 + k * CB, CB)])
        pltpu.sync_copy(pb0, den_sh.at[pl.ds(row0 + k * CB, CB)])
    plsc.subcore_barrier()

    bufs = ((hr0, wr0, pb0, gsem0, nsem0, dsem0),
            (hr1, wr1, pb1, gsem1, nsem1, dsem1))

    def _compute(k, hr, wr, pb):
        for g in range(CB // L):
            siv = idx_v[k, pl.ds(g * L, L)]
            div = idx_v[CHUNKS + k, pl.ds(g * L, L)]
            av = plsc.load_gather(as_v, [siv])
            dv = plsc.load_gather(ad_v, [div])
            s = av + dv
            e = jnp.maximum(s, 0.2 * s)
            p = jnp.exp(e - cval)
            pb[pl.ds(g * L, L)] = p
            for j in range(L):
                pi = p[j]
                wr[g * L + j, :] = hr[g * L + j, :] * pi

    pltpu.async_copy(h_hbm.at[idx_v.at[0]], hr0, gsem0)
    pltpu.async_copy(h_hbm.at[idx_v.at[1]], hr1, gsem1)

    def _pair(i, carry):
        for b in range(2):
            hr, wr, pb, gsem, nsem, dsem = bufs[b]
            k = 2 * i + b
            pltpu.make_async_copy(h_hbm.at[idx_v.at[k]], hr, gsem).wait()

            @pl.when(i > 0)
            def _():
                dk = idx_v.at[CHUNKS + k - 2]
                pltpu.make_async_copy(wr, num_sh.at[dk], nsem).wait()
                pltpu.make_async_copy(pb, den_sh.at[dk], dsem).wait()

            _compute(k, hr, wr, pb)
            dk = idx_v.at[CHUNKS + k]
            pltpu.async_copy(wr, num_sh.at[dk], nsem, add=True)
            pltpu.async_copy(pb, den_sh.at[dk], dsem, add=True)

            @pl.when(k + 2 < CHUNKS)
            def _():
                pltpu.async_copy(h_hbm.at[idx_v.at[k + 2]], hr, gsem)
        return carry

    lax.fori_loop(0, CHUNKS // 2, _pair, 0)
    for b in range(2):
        hr, wr, pb, gsem, nsem, dsem = bufs[b]
        dk = idx_v.at[2 * CHUNKS - 2 + b]
        pltpu.make_async_copy(wr, num_sh.at[dk], nsem).wait()
        pltpu.make_async_copy(pb, den_sh.at[dk], dsem).wait()
    plsc.subcore_barrier()

    for k in range(ROWS_PER // CB):
        r = row0 + k * CB
        pltpu.sync_copy(num_sh.at[pl.ds(r, CB)], wr0)
        pltpu.sync_copy(wr0, num_out.at[cid].at[pl.ds(r, CB)])
        pltpu.sync_copy(den_sh.at[pl.ds(r, CB)], pb0)
        pltpu.sync_copy(pb0, den_out.at[cid].at[pl.ds(r, CB)])


_sc_edges = pl.kernel(
    _sc_edges_body,
    out_type=(
        jax.ShapeDtypeStruct((NC, NP, F), jnp.float32),
        jax.ShapeDtypeStruct((NC, NP), jnp.float32),
    ),
    mesh=plsc.VectorSubcoreMesh(core_axis_name="c", subcore_axis_name="s",
                                num_cores=NC, num_subcores=NS),
    scratch_types=[
        pltpu.VMEM((NP,), jnp.float32),
        pltpu.VMEM((NP,), jnp.float32),
        pltpu.VMEM((L,), jnp.float32),
        pltpu.VMEM((2 * CHUNKS, CB), jnp.int32),
        pltpu.VMEM((CB, F), jnp.float32),
        pltpu.VMEM((CB, F), jnp.float32),
        pltpu.VMEM((CB, F), jnp.float32),
        pltpu.VMEM((CB, F), jnp.float32),
        pltpu.VMEM((CB,), jnp.float32),
        pltpu.VMEM((CB,), jnp.float32),
        pltpu.VMEM_SHARED((NP, F), jnp.float32),
        pltpu.VMEM_SHARED((NP,), jnp.float32),
        pltpu.SemaphoreType.DMA,
        pltpu.SemaphoreType.DMA,
        pltpu.SemaphoreType.DMA,
        pltpu.SemaphoreType.DMA,
        pltpu.SemaphoreType.DMA,
        pltpu.SemaphoreType.DMA,
    ],
    compiler_params=pltpu.CompilerParams(needs_layout_passes=False,
                                         use_tc_tiling_on_sc=False),
)


def kernel(x, edge_index, W1, a1s, a1d, b1, W2, a2s, a2d, b2,
           W3, a3s, a3d, b3, Wl1, bl1, Wl2, bl2, Wl3, bl3, Wl4, bl4):
    loop = jnp.arange(N, dtype=jnp.int32)
    padi = jnp.full((EPAD - ESL,), N, dtype=jnp.int32)
    srcp = jnp.concatenate([edge_index[0], loop, padi]).reshape(NW, CHUNKS, CB)
    dstp = jnp.concatenate([edge_index[1], loop, padi]).reshape(NW, CHUNKS, CB)
    idx3 = jnp.concatenate([srcp, dstp], axis=1)
    x_pad = jnp.pad(x, ((0, NP - N), (0, 0)))

    h, asv, adv, cvec = _tc_front(x_pad, W1, a1s, a1d)
    numP, denP = _sc_edges(idx3, h, asv, adv, cvec)
    h, asv, adv, cvec = _tc_mid(numP, denP, b1, W2, a2s, a2d)
    numP, denP = _sc_edges(idx3, h, asv, adv, cvec)
    h, asv, adv, cvec = _tc_mid(numP, denP, b2, W3, a3s, a3d)
    numP, denP = _sc_edges(idx3, h, asv, adv, cvec)
    out = _tc_final(numP, denP, b3, Wl1, bl1, Wl2, bl2, Wl3, bl3, Wl4, bl4)
    return out[:N]
